# Initial kernel scaffold; baseline (speedup 1.0000x reference)
import jax
import jax.numpy as jnp
from jax.experimental import pallas as pl


def kernel(x, edge_index, edge_weight, params):
    raise NotImplementedError("write your pallas kernel here")


# speedup vs baseline: 19.7937x; 19.7937x over previous
import functools

import jax
import jax.numpy as jnp
from jax import lax
from jax.experimental import pallas as pl
from jax.experimental.pallas import tpu as pltpu
from jax.experimental.pallas import tpu_sc as plsc

N = 10000
E = 320000
F = 128
T = 12
H = 64
OUT = 128

NC = 2
NS = 16
LANES = 16

CE = E // NS
KB = 80
NB = CE // KB
ROWS_PER_TILE = 624
ZROWS = 16
SROWS = 208
REM_OFF = ROWS_PER_TILE * NS

ED = E // (NC * NS)
KD = 125
NBD = ED // KD


def _sc_mesh():
    return plsc.VectorSubcoreMesh(core_axis_name="c", subcore_axis_name="s",
                                  num_cores=NC, num_subcores=NS)


def _deg_body(dst_hbm, ew_hbm, degp_hbm, dstv, ewv, acc, zbuf):
    c = lax.axis_index("c")
    s = lax.axis_index("s")
    b = c * NS + s
    pltpu.sync_copy(dst_hbm.at[b], dstv)
    pltpu.sync_copy(ew_hbm.at[b], ewv)

    def zz(i, _):
        zbuf[pl.ds(i * LANES, LANES)] = jnp.zeros((LANES,), jnp.float32)
        return 0
    lax.fori_loop(0, 2000 // LANES, zz, 0)

    @pl.when(s < 5)
    def _():
        pltpu.sync_copy(zbuf, acc.at[pl.ds(s * 2000, 2000)])
    plsc.subcore_barrier()

    def body(bb, _):
        pltpu.sync_copy(ewv.at[bb], acc.at[dstv.at[bb]], add=True)
        return 0
    lax.fori_loop(0, NBD, body, 0)
    plsc.subcore_barrier()

    @pl.when(s < 5)
    def _():
        pltpu.sync_copy(acc.at[pl.ds(s * 2000, 2000)], zbuf)
        pltpu.sync_copy(zbuf, degp_hbm.at[pl.ds(c * N + s * 2000, 2000)])


def _make_deg_kernel():
    return pl.kernel(
        _deg_body,
        out_type=jax.ShapeDtypeStruct((NC * N,), jnp.float32),
        mesh=_sc_mesh(),
        scratch_types=[
            pltpu.VMEM((NBD, KD), jnp.int32),
            pltpu.VMEM((NBD, KD), jnp.float32),
            pltpu.VMEM_SHARED((N,), jnp.float32),
            pltpu.VMEM((2000,), jnp.float32),
        ],
    )


def _spmm_body(W, TCH, y_hbm, src_hbm, dst_hbm, ew_hbm, p_hbm,
               srcv, dstv, ewv, idxv, rows, zbuf, sbuf, acc):
    c = lax.axis_index("c")
    s = lax.axis_index("s")
    pltpu.sync_copy(src_hbm.at[s], srcv)
    pltpu.sync_copy(dst_hbm.at[s], dstv)
    pltpu.sync_copy(ew_hbm.at[s], ewv)

    def zz(i, _):
        for j in range(W // LANES):
            zbuf[i, pl.ds(j * LANES, LANES)] = jnp.zeros((LANES,), jnp.float32)
        return 0
    lax.fori_loop(0, ZROWS, zz, 0)

    def chunk(jt, _):
        t = 2 * jt + c

        def zacc(k, _):
            pltpu.sync_copy(zbuf, acc.at[pl.ds(s * ROWS_PER_TILE + k * ZROWS,
                                               ZROWS), :])
            return 0
        lax.fori_loop(0, ROWS_PER_TILE // ZROWS, zacc, 0)

        @pl.when(s == NS - 1)
        def _():
            pltpu.sync_copy(zbuf, acc.at[pl.ds(REM_OFF, N - REM_OFF), :])
        plsc.subcore_barrier()

        tn = jnp.full((LANES,), t * N, jnp.int32)

        def batch(b, _):
            base = b * KB
            for j in range(KB // LANES):
                idxv[pl.ds(j * LANES, LANES)] = (
                    srcv[pl.ds(base + j * LANES, LANES)] + tn)
            pltpu.sync_copy(y_hbm.at[idxv], rows)

            for j in range(KB // LANES):
                nvec = ewv[pl.ds(base + j * LANES, LANES)]
                for i in range(LANES):
                    nv = lax.broadcast_in_dim(nvec[i], (LANES,), ())
                    r = j * LANES + i
                    for q in range(W // LANES):
                        rows[r, pl.ds(q * LANES, LANES)] = (
                            rows[r, pl.ds(q * LANES, LANES)] * nv)
            pltpu.sync_copy(rows, acc.at[dstv.at[b]], add=True)
            return 0
        lax.fori_loop(0, NB, batch, 0)
        plsc.subcore_barrier()

        for k in range(ROWS_PER_TILE // SROWS):
            r0 = s * ROWS_PER_TILE + k * SROWS
            pltpu.sync_copy(acc.at[pl.ds(r0, SROWS), :], sbuf)
            pltpu.sync_copy(sbuf, p_hbm.at[t, pl.ds(r0, SROWS), :])

        @pl.when(s == NS - 1)
        def _():
            nrem = N - REM_OFF
            pltpu.sync_copy(acc.at[pl.ds(REM_OFF, nrem), :],
                            sbuf.at[pl.ds(0, nrem), :])
            pltpu.sync_copy(sbuf.at[pl.ds(0, nrem), :],
                            p_hbm.at[t, pl.ds(REM_OFF, nrem), :])
        plsc.subcore_barrier()
        return 0
    lax.fori_loop(0, TCH // NC, chunk, 0)


def _make_spmm_kernel(TCH, W=H):
    return pl.kernel(
        functools.partial(_spmm_body, W, TCH),
        out_type=jax.ShapeDtypeStruct((TCH, N, W), jnp.float32),
        mesh=_sc_mesh(),
        scratch_types=[
            pltpu.VMEM((CE,), jnp.int32),
            pltpu.VMEM((NB, KB), jnp.int32),
            pltpu.VMEM((CE,), jnp.float32),
            pltpu.VMEM((KB,), jnp.int32),
            pltpu.VMEM((KB, W), jnp.float32),
            pltpu.VMEM((ZROWS, W), jnp.float32),
            pltpu.VMEM((SROWS, W), jnp.float32),
            pltpu.VMEM_SHARED((N, W), jnp.float32),
        ],
        compiler_params=pltpu.CompilerParams(use_tc_tiling_on_sc=False),
    )


_make_deg_kernel = functools.lru_cache(None)(_make_deg_kernel)
_make_spmm_kernel = functools.lru_cache(None)(_make_spmm_kernel)


def _deg_kernel(dst3, ew3):
    return _make_deg_kernel()(dst3, ew3)


def _spmm_f(*args):
    return _make_spmm_kernel(2 * T)(*args)


def _spmm_h(*args):
    return _make_spmm_kernel(T)(*args)


def _dinv_body(degp_ref, dinv_ref):
    deg = degp_ref[pl.ds(0, N)] + degp_ref[pl.ds(N, N)] + 1.0
    dinv_ref[...] = lax.rsqrt(jnp.maximum(deg, 1e-12))


def _dinv_kernel(degp):
    return pl.pallas_call(
        _dinv_body,
        out_shape=jax.ShapeDtypeStruct((N,), jnp.float32),
    )(degp)


def _prescale_body(x_ref, dinv_ref, y_ref):
    xx = x_ref[...] * dinv_ref[...][None]
    y_ref[:, 0] = xx[:, :, :H]
    y_ref[:, 1] = xx[:, :, H:]


def _prescale(X, dinv2, bn=1000):
    return pl.pallas_call(
        _prescale_body,
        grid=(N // bn,),
        in_specs=[pl.BlockSpec((T, bn, F), lambda i: (0, i, 0)),
                  pl.BlockSpec((bn, 1), lambda i: (i, 0))],
        out_specs=pl.BlockSpec((T, 2, bn, H), lambda i: (0, 0, i, 0)),
        out_shape=jax.ShapeDtypeStruct((T, 2, N, H), jnp.float32),
    )(X, dinv2)


def _layer_body(with_head, split, z_ref, y_ref, dinv_ref, wc_ref, bc_ref,
                wlt_ref, wlbzr_ref, wlbh_ref, blzr_ref, blh_ref,
                *rest):
    if with_head:
        (watt_ref, batt_ref, wout_ref, bout_ref, out_ref) = rest
    else:
        (out_ref,) = rest
    bn = dinv_ref.shape[0]
    dinv = dinv_ref[...]
    h = jnp.zeros((bn, H), jnp.float32)
    wc = wc_ref[...]
    bc = bc_ref[...]
    wlt = wlt_ref[...]
    wlbzr = wlbzr_ref[...]
    wlbh = wlbh_ref[...]
    blzr = blzr_ref[...]
    blh = blh_ref[...]
    hs = []
    es = []
    for t in range(T):
        if split:
            zt = jnp.concatenate([z_ref[2 * t], z_ref[2 * t + 1]], axis=1)
            yt = jnp.concatenate([y_ref[t, 0], y_ref[t, 1]], axis=1)
        else:
            zt = z_ref[t]
            yt = y_ref[t]
        conv = dinv * (zt + yt)
        c3 = jnp.dot(conv, wc, preferred_element_type=jnp.float32) + bc
        p3 = jnp.dot(c3, wlt, preferred_element_type=jnp.float32)
        hzr = jnp.dot(h, wlbzr, preferred_element_type=jnp.float32)
        zr = jax.nn.sigmoid(p3[:, :2 * H] + hzr + blzr)
        z = zr[:, :H]
        r = zr[:, H:]
        htil = jnp.tanh(p3[:, 2 * H:] +
                        jnp.dot(h * r, wlbh, preferred_element_type=jnp.float32)
                        + blh)
        h = z * h + (1.0 - z) * htil
        if with_head:
            hs.append(h)
            es.append(jnp.sum(h * watt_ref[...], axis=1) + batt_ref[0])
        else:
            out_ref[t] = h * dinv
    if with_head:
        e = jnp.stack(es, axis=0)
        m = jnp.max(e, axis=0, keepdims=True)
        pe = jnp.exp(e - m)
        att = pe / jnp.sum(pe, axis=0, keepdims=True)
        ctx = jnp.zeros((bn, H), jnp.float32)
        for t in range(T):
            ctx = ctx + att[t][:, None] * hs[t]
        out_ref[...] = (jnp.dot(ctx, wout_ref[...],
                                preferred_element_type=jnp.float32)
                        + bout_ref[...])


def _run_layer(Z, Y, dinv2, wc, bc, wlt, wlbzr, wlbh, blzr, blh,
               head=None, split=False, bn=400):
    grid = (N // bn,)
    full = lambda shape: pl.BlockSpec(shape, lambda i: (0,) * len(shape))
    if split:
        zspec = pl.BlockSpec((2 * T, bn, H), lambda i: (0, i, 0))
        yspec = pl.BlockSpec((T, 2, bn, H), lambda i: (0, 0, i, 0))
    else:
        zspec = pl.BlockSpec((T, bn, H), lambda i: (0, i, 0))
        yspec = pl.BlockSpec((T, bn, H), lambda i: (0, i, 0))
    in_specs = [
        zspec, yspec,
        pl.BlockSpec((bn, 1), lambda i: (i, 0)),
        full(wc.shape), full(bc.shape), full(wlt.shape),
        full(wlbzr.shape), full(wlbh.shape), full(blzr.shape),
        full(blh.shape),
    ]
    args = [Z, Y, dinv2, wc, bc, wlt, wlbzr, wlbh, blzr, blh]
    if head is None:
        out_shape = jax.ShapeDtypeStruct((T, N, H), jnp.float32)
        out_spec = pl.BlockSpec((T, bn, H), lambda i: (0, i, 0))
        body = functools.partial(_layer_body, False, split)
    else:
        watt, batt, wout, bout = head
        in_specs += [full(watt.shape), full(batt.shape), full(wout.shape),
                     full(bout.shape)]
        args += [watt, batt, wout, bout]
        out_shape = jax.ShapeDtypeStruct((N, OUT), jnp.float32)
        out_spec = pl.BlockSpec((bn, OUT), lambda i: (i, 0))
        body = functools.partial(_layer_body, True, split)
    return pl.pallas_call(
        body, grid=grid, in_specs=in_specs, out_specs=out_spec,
        out_shape=out_shape,
    )(*args)


def _gate_weights(lp):
    wc = jnp.concatenate([lp["Wc_z"], lp["Wc_r"], lp["Wc_h"]], axis=1)
    bc = jnp.concatenate([lp["bc_z"], lp["bc_r"], lp["bc_h"]])
    wlt = jax.scipy.linalg.block_diag(lp["Wl_z"][:H], lp["Wl_r"][:H],
                                      lp["Wl_h"][:H])
    wlbzr = jnp.concatenate([lp["Wl_z"][H:], lp["Wl_r"][H:]], axis=1)
    wlbh = lp["Wl_h"][H:]
    blzr = jnp.concatenate([lp["bl_z"], lp["bl_r"]])
    blh = lp["bl_h"]
    return wc, bc, wlt, wlbzr, wlbh, blzr, blh


def kernel(x, edge_index, edge_weight, params):
    src = edge_index[0].astype(jnp.int32)
    dst = edge_index[1].astype(jnp.int32)
    ew = edge_weight.astype(jnp.float32)

    degp = _deg_kernel(dst.reshape(NC * NS, NBD, KD),
                       ew.reshape(NC * NS, NBD, KD))
    dinv = _dinv_kernel(degp)
    dinv2 = dinv.reshape(N, 1)

    src2 = src.reshape(NS, CE)
    dst3 = dst.reshape(NS, NB, KB)
    ew2 = ew.reshape(NS, CE)

    xt = jnp.transpose(x, (2, 0, 1))
    Y0 = _prescale(xt, dinv2)
    Z1 = _spmm_f(Y0.reshape(2 * T * N, H), src2, dst3, ew2)

    w0 = _gate_weights(params["layers"][0])
    Y1 = _run_layer(Z1, Y0, dinv2, *w0, split=True)

    Z2 = _spmm_h(Y1.reshape(T * N, H), src2, dst3, ew2)

    w1 = _gate_weights(params["layers"][1])
    head = (params["W_att"].reshape(1, H), params["b_att"],
            params["W_out"], params["b_out"])
    return _run_layer(Z2, Y1, dinv2, *w1, head=head)


# speedup vs baseline: 35.0382x; 1.7702x over previous
import functools

import jax
import jax.numpy as jnp
from jax import lax
from jax.experimental import pallas as pl
from jax.experimental.pallas import tpu as pltpu
from jax.experimental.pallas import tpu_sc as plsc

N = 10000
E = 320000
F = 128
T = 12
H = 64
OUT = 128

NC = 2
NS = 16
LANES = 16

CE = E // NS
KB = 80
NB = CE // KB
ROWS_PER_TILE = 624
ZROWS = 104
SROWS = 208
REM_OFF = ROWS_PER_TILE * NS

ED = E // (NC * NS)
KD = 125
NBD = ED // KD


def _sc_mesh():
    return plsc.VectorSubcoreMesh(core_axis_name="c", subcore_axis_name="s",
                                  num_cores=NC, num_subcores=NS)


def _deg_body(dst_hbm, ew_hbm, degp_hbm, dstv, ewv, acc, zbuf):
    c = lax.axis_index("c")
    s = lax.axis_index("s")
    b = c * NS + s
    pltpu.sync_copy(dst_hbm.at[b], dstv)
    pltpu.sync_copy(ew_hbm.at[b], ewv)

    def zz(i, _):
        zbuf[pl.ds(i * LANES, LANES)] = jnp.zeros((LANES,), jnp.float32)
        return 0
    lax.fori_loop(0, 2000 // LANES, zz, 0)

    @pl.when(s < 5)
    def _():
        pltpu.sync_copy(zbuf, acc.at[pl.ds(s * 2000, 2000)])
    plsc.subcore_barrier()

    def body(bb, _):
        pltpu.sync_copy(ewv.at[bb], acc.at[dstv.at[bb]], add=True)
        return 0
    lax.fori_loop(0, NBD, body, 0)
    plsc.subcore_barrier()

    @pl.when(s < 5)
    def _():
        pltpu.sync_copy(acc.at[pl.ds(s * 2000, 2000)], zbuf)
        pltpu.sync_copy(zbuf, degp_hbm.at[pl.ds(c * N + s * 2000, 2000)])


def _make_deg_kernel():
    return pl.kernel(
        _deg_body,
        out_type=jax.ShapeDtypeStruct((NC * N,), jnp.float32),
        mesh=_sc_mesh(),
        scratch_types=[
            pltpu.VMEM((NBD, KD), jnp.int32),
            pltpu.VMEM((NBD, KD), jnp.float32),
            pltpu.VMEM_SHARED((N,), jnp.float32),
            pltpu.VMEM((2000,), jnp.float32),
        ],
    )


def _spmm_body(W, TCH, y_hbm, src_hbm, dst_hbm, ew_hbm, p_hbm,
               srcv, dstv, ewv, idx0, idx1, rows0, rows1, zbuf, sbuf, acc,
               gsem0, gsem1):
    c = lax.axis_index("c")
    s = lax.axis_index("s")
    pltpu.sync_copy(src_hbm.at[s], srcv)
    pltpu.sync_copy(dst_hbm.at[s], dstv)
    pltpu.sync_copy(ew_hbm.at[s], ewv)

    def zz(i, _):
        for j in range(W // LANES):
            zbuf[i, pl.ds(j * LANES, LANES)] = jnp.zeros((LANES,), jnp.float32)
        return 0
    lax.fori_loop(0, ZROWS, zz, 0)

    bufs = ((idx0, rows0, gsem0), (idx1, rows1, gsem1))

    def chunk(jt, _):
        t = 2 * jt + c

        for k in range(ROWS_PER_TILE // ZROWS):
            pltpu.sync_copy(zbuf, acc.at[pl.ds(s * ROWS_PER_TILE + k * ZROWS,
                                               ZROWS), :])

        @pl.when(s == NS - 1)
        def _():
            nrem = N - REM_OFF
            pltpu.sync_copy(zbuf.at[pl.ds(0, nrem), :],
                            acc.at[pl.ds(REM_OFF, nrem), :])
        plsc.subcore_barrier()

        tn = jnp.full((LANES,), t * N, jnp.int32)

        def gstart(b, ibuf, rbuf, sem):
            base = b * KB
            for j in range(KB // LANES):
                ibuf[pl.ds(j * LANES, LANES)] = (
                    srcv[pl.ds(base + j * LANES, LANES)] + tn)
            pltpu.async_copy(y_hbm.at[ibuf], rbuf, sem)

        gstart(0, *bufs[0])
        gstart(1, *bufs[1])

        def pair(pj, _):
            for p in range(2):
                ibuf, rbuf, sem = bufs[p]
                b = 2 * pj + p
                base = b * KB
                pltpu.make_async_copy(y_hbm.at[ibuf], rbuf, sem).wait()
                for j in range(KB // LANES):
                    nvec = ewv[pl.ds(base + j * LANES, LANES)]
                    for i in range(LANES):
                        nv = lax.broadcast_in_dim(nvec[i], (LANES,), ())
                        r = j * LANES + i
                        for q in range(W // LANES):
                            rbuf[r, pl.ds(q * LANES, LANES)] = (
                                rbuf[r, pl.ds(q * LANES, LANES)] * nv)
                pltpu.sync_copy(rbuf, acc.at[dstv.at[b]], add=True)

                @pl.when(b + 2 < NB)
                def _():
                    gstart(b + 2, ibuf, rbuf, sem)
            return 0
        lax.fori_loop(0, NB // 2, pair, 0)
        plsc.subcore_barrier()

        for k in range(ROWS_PER_TILE // SROWS):
            r0 = s * ROWS_PER_TILE + k * SROWS
            pltpu.sync_copy(acc.at[pl.ds(r0, SROWS), :], sbuf)
            pltpu.sync_copy(sbuf, p_hbm.at[t, pl.ds(r0, SROWS), :])

        @pl.when(s == NS - 1)
        def _():
            nrem = N - REM_OFF
            pltpu.sync_copy(acc.at[pl.ds(REM_OFF, nrem), :],
                            sbuf.at[pl.ds(0, nrem), :])
            pltpu.sync_copy(sbuf.at[pl.ds(0, nrem), :],
                            p_hbm.at[t, pl.ds(REM_OFF, nrem), :])
        plsc.subcore_barrier()
        return 0
    lax.fori_loop(0, TCH // NC, chunk, 0)


def _make_spmm_kernel(TCH, W=H):
    return pl.kernel(
        functools.partial(_spmm_body, W, TCH),
        out_type=jax.ShapeDtypeStruct((TCH, N, W), jnp.float32),
        mesh=_sc_mesh(),
        scratch_types=[
            pltpu.VMEM((CE,), jnp.int32),
            pltpu.VMEM((NB, KB), jnp.int32),
            pltpu.VMEM((CE,), jnp.float32),
            pltpu.VMEM((KB,), jnp.int32),
            pltpu.VMEM((KB,), jnp.int32),
            pltpu.VMEM((KB, W), jnp.float32),
            pltpu.VMEM((KB, W), jnp.float32),
            pltpu.VMEM((ZROWS, W), jnp.float32),
            pltpu.VMEM((SROWS, W), jnp.float32),
            pltpu.VMEM_SHARED((N, W), jnp.float32),
            pltpu.SemaphoreType.DMA,
            pltpu.SemaphoreType.DMA,
        ],
        compiler_params=pltpu.CompilerParams(use_tc_tiling_on_sc=False),
    )


_make_deg_kernel = functools.lru_cache(None)(_make_deg_kernel)
_make_spmm_kernel = functools.lru_cache(None)(_make_spmm_kernel)


def _deg_kernel(dst3, ew3):
    return _make_deg_kernel()(dst3, ew3)


def _spmm_f(*args):
    return _make_spmm_kernel(2 * T)(*args)


def _spmm_h(*args):
    return _make_spmm_kernel(T)(*args)


def _dinv_body(degp_ref, dinv_ref):
    deg = degp_ref[pl.ds(0, N)] + degp_ref[pl.ds(N, N)] + 1.0
    dinv_ref[...] = lax.rsqrt(jnp.maximum(deg, 1e-12))


def _dinv_kernel(degp):
    return pl.pallas_call(
        _dinv_body,
        out_shape=jax.ShapeDtypeStruct((N,), jnp.float32),
    )(degp)


def _prescale_body(x_ref, dinv_ref, y_ref):
    xx = x_ref[...] * dinv_ref[...][None]
    y_ref[:, 0] = xx[:, :, :H]
    y_ref[:, 1] = xx[:, :, H:]


def _prescale(X, dinv2, bn=1000):
    return pl.pallas_call(
        _prescale_body,
        grid=(N // bn,),
        in_specs=[pl.BlockSpec((T, bn, F), lambda i: (0, i, 0)),
                  pl.BlockSpec((bn, 1), lambda i: (i, 0))],
        out_specs=pl.BlockSpec((T, 2, bn, H), lambda i: (0, 0, i, 0)),
        out_shape=jax.ShapeDtypeStruct((T, 2, N, H), jnp.float32),
    )(X, dinv2)


def _layer_body(with_head, split, z_ref, y_ref, dinv_ref, wc_ref, bc_ref,
                wlt_ref, wlbzr_ref, wlbh_ref, blzr_ref, blh_ref,
                *rest):
    if with_head:
        (watt_ref, batt_ref, wout_ref, bout_ref, out_ref) = rest
    else:
        (out_ref,) = rest
    bn = dinv_ref.shape[0]
    dinv = dinv_ref[...]
    h = jnp.zeros((bn, H), jnp.float32)
    wc = wc_ref[...]
    bc = bc_ref[...]
    wlt = wlt_ref[...]
    wlbzr = wlbzr_ref[...]
    wlbh = wlbh_ref[...]
    blzr = blzr_ref[...]
    blh = blh_ref[...]
    hs = []
    es = []
    for t in range(T):
        if split:
            zt = jnp.concatenate([z_ref[2 * t], z_ref[2 * t + 1]], axis=1)
            yt = jnp.concatenate([y_ref[t, 0], y_ref[t, 1]], axis=1)
        else:
            zt = z_ref[t]
            yt = y_ref[t]
        conv = dinv * (zt + yt)
        c3 = jnp.dot(conv, wc, preferred_element_type=jnp.float32) + bc
        p3 = jnp.dot(c3, wlt, preferred_element_type=jnp.float32)
        hzr = jnp.dot(h, wlbzr, preferred_element_type=jnp.float32)
        zr = jax.nn.sigmoid(p3[:, :2 * H] + hzr + blzr)
        z = zr[:, :H]
        r = zr[:, H:]
        htil = jnp.tanh(p3[:, 2 * H:] +
                        jnp.dot(h * r, wlbh, preferred_element_type=jnp.float32)
                        + blh)
        h = z * h + (1.0 - z) * htil
        if with_head:
            hs.append(h)
            es.append(jnp.sum(h * watt_ref[...], axis=1) + batt_ref[0])
        else:
            out_ref[t] = h * dinv
    if with_head:
        e = jnp.stack(es, axis=0)
        m = jnp.max(e, axis=0, keepdims=True)
        pe = jnp.exp(e - m)
        att = pe / jnp.sum(pe, axis=0, keepdims=True)
        ctx = jnp.zeros((bn, H), jnp.float32)
        for t in range(T):
            ctx = ctx + att[t][:, None] * hs[t]
        out_ref[...] = (jnp.dot(ctx, wout_ref[...],
                                preferred_element_type=jnp.float32)
                        + bout_ref[...])


def _run_layer(Z, Y, dinv2, wc, bc, wlt, wlbzr, wlbh, blzr, blh,
               head=None, split=False, bn=400):
    grid = (N // bn,)
    full = lambda shape: pl.BlockSpec(shape, lambda i: (0,) * len(shape))
    if split:
        zspec = pl.BlockSpec((2 * T, bn, H), lambda i: (0, i, 0))
        yspec = pl.BlockSpec((T, 2, bn, H), lambda i: (0, 0, i, 0))
    else:
        zspec = pl.BlockSpec((T, bn, H), lambda i: (0, i, 0))
        yspec = pl.BlockSpec((T, bn, H), lambda i: (0, i, 0))
    in_specs = [
        zspec, yspec,
        pl.BlockSpec((bn, 1), lambda i: (i, 0)),
        full(wc.shape), full(bc.shape), full(wlt.shape),
        full(wlbzr.shape), full(wlbh.shape), full(blzr.shape),
        full(blh.shape),
    ]
    args = [Z, Y, dinv2, wc, bc, wlt, wlbzr, wlbh, blzr, blh]
    if head is None:
        out_shape = jax.ShapeDtypeStruct((T, N, H), jnp.float32)
        out_spec = pl.BlockSpec((T, bn, H), lambda i: (0, i, 0))
        body = functools.partial(_layer_body, False, split)
    else:
        watt, batt, wout, bout = head
        in_specs += [full(watt.shape), full(batt.shape), full(wout.shape),
                     full(bout.shape)]
        args += [watt, batt, wout, bout]
        out_shape = jax.ShapeDtypeStruct((N, OUT), jnp.float32)
        out_spec = pl.BlockSpec((bn, OUT), lambda i: (i, 0))
        body = functools.partial(_layer_body, True, split)
    return pl.pallas_call(
        body, grid=grid, in_specs=in_specs, out_specs=out_spec,
        out_shape=out_shape,
    )(*args)


def _gate_weights(lp):
    wc = jnp.concatenate([lp["Wc_z"], lp["Wc_r"], lp["Wc_h"]], axis=1)
    bc = jnp.concatenate([lp["bc_z"], lp["bc_r"], lp["bc_h"]])
    wlt = jax.scipy.linalg.block_diag(lp["Wl_z"][:H], lp["Wl_r"][:H],
                                      lp["Wl_h"][:H])
    wlbzr = jnp.concatenate([lp["Wl_z"][H:], lp["Wl_r"][H:]], axis=1)
    wlbh = lp["Wl_h"][H:]
    blzr = jnp.concatenate([lp["bl_z"], lp["bl_r"]])
    blh = lp["bl_h"]
    return wc, bc, wlt, wlbzr, wlbh, blzr, blh


def kernel(x, edge_index, edge_weight, params):
    src = edge_index[0].astype(jnp.int32)
    dst = edge_index[1].astype(jnp.int32)
    ew = edge_weight.astype(jnp.float32)

    degp = _deg_kernel(dst.reshape(NC * NS, NBD, KD),
                       ew.reshape(NC * NS, NBD, KD))
    dinv = _dinv_kernel(degp)
    dinv2 = dinv.reshape(N, 1)

    src2 = src.reshape(NS, CE)
    dst3 = dst.reshape(NS, NB, KB)
    ew2 = ew.reshape(NS, CE)

    xt = jnp.transpose(x, (2, 0, 1))
    Y0 = _prescale(xt, dinv2)
    Z1 = _spmm_f(Y0.reshape(2 * T * N, H), src2, dst3, ew2)

    w0 = _gate_weights(params["layers"][0])
    Y1 = _run_layer(Z1, Y0, dinv2, *w0, split=True)

    Z2 = _spmm_h(Y1.reshape(T * N, H), src2, dst3, ew2)

    w1 = _gate_weights(params["layers"][1])
    head = (params["W_att"].reshape(1, H), params["b_att"],
            params["W_out"], params["b_out"])
    return _run_layer(Z2, Y1, dinv2, *w1, head=head)


# speedup vs baseline: 40.4360x; 1.1541x over previous
import functools

import jax
import jax.numpy as jnp
from jax import lax
from jax.experimental import pallas as pl
from jax.experimental.pallas import tpu as pltpu
from jax.experimental.pallas import tpu_sc as plsc

N = 10000
E = 320000
F = 128
T = 12
H = 64
OUT = 128

NC = 2
NS = 16
LANES = 16

CE = E // NS
KB = 80
NB = CE // KB
ROWS_PER_TILE = 624
ZROWS = 52
SROWS = 104
REM_OFF = ROWS_PER_TILE * NS

ED = E // (NC * NS)
KD = 125
NBD = ED // KD


def _sc_mesh():
    return plsc.VectorSubcoreMesh(core_axis_name="c", subcore_axis_name="s",
                                  num_cores=NC, num_subcores=NS)


def _deg_body(dst_hbm, ew_hbm, degp_hbm, dstv, ewv, acc, zbuf):
    c = lax.axis_index("c")
    s = lax.axis_index("s")
    b = c * NS + s
    pltpu.sync_copy(dst_hbm.at[b], dstv)
    pltpu.sync_copy(ew_hbm.at[b], ewv)

    def zz(i, _):
        zbuf[pl.ds(i * LANES, LANES)] = jnp.zeros((LANES,), jnp.float32)
        return 0
    lax.fori_loop(0, 2000 // LANES, zz, 0)

    @pl.when(s < 5)
    def _():
        pltpu.sync_copy(zbuf, acc.at[pl.ds(s * 2000, 2000)])
    plsc.subcore_barrier()

    def body(bb, _):
        pltpu.sync_copy(ewv.at[bb], acc.at[dstv.at[bb]], add=True)
        return 0
    lax.fori_loop(0, NBD, body, 0)
    plsc.subcore_barrier()

    @pl.when(s < 5)
    def _():
        pltpu.sync_copy(acc.at[pl.ds(s * 2000, 2000)], zbuf)
        pltpu.sync_copy(zbuf, degp_hbm.at[pl.ds(c * N + s * 2000, 2000)])


def _make_deg_kernel():
    return pl.kernel(
        _deg_body,
        out_type=jax.ShapeDtypeStruct((NC * N,), jnp.float32),
        mesh=_sc_mesh(),
        scratch_types=[
            pltpu.VMEM((NBD, KD), jnp.int32),
            pltpu.VMEM((NBD, KD), jnp.float32),
            pltpu.VMEM_SHARED((N,), jnp.float32),
            pltpu.VMEM((2000,), jnp.float32),
        ],
    )


def _spmm_body(W, TCH, y_hbm, src_hbm, dst_hbm, ew_hbm, p_hbm,
               srcv, dstv, ewv, idx0, idx1, rows0, rows1, sout0, sout1,
               zbuf, sbuf, acc, gsem0, gsem1, ssem0, ssem1):
    c = lax.axis_index("c")
    s = lax.axis_index("s")
    pltpu.sync_copy(src_hbm.at[s], srcv)
    pltpu.sync_copy(dst_hbm.at[s], dstv)
    pltpu.sync_copy(ew_hbm.at[s], ewv)

    def zz(i, _):
        for j in range(W // LANES):
            zbuf[i, pl.ds(j * LANES, LANES)] = jnp.zeros((LANES,), jnp.float32)
        return 0
    lax.fori_loop(0, ZROWS, zz, 0)

    bufs = ((idx0, rows0, gsem0, sout0, ssem0),
            (idx1, rows1, gsem1, sout1, ssem1))

    def chunk(jt, _):
        t = 2 * jt + c

        for k in range(ROWS_PER_TILE // ZROWS):
            pltpu.sync_copy(zbuf, acc.at[pl.ds(s * ROWS_PER_TILE + k * ZROWS,
                                               ZROWS), :])

        @pl.when(s == NS - 1)
        def _():
            nrem = N - REM_OFF
            pltpu.sync_copy(zbuf.at[pl.ds(0, nrem), :],
                            acc.at[pl.ds(REM_OFF, nrem), :])
        plsc.subcore_barrier()

        tn = jnp.full((LANES,), t * N, jnp.int32)

        def gstart(b, ibuf, rbuf, sem):
            base = b * KB
            for j in range(KB // LANES):
                ibuf[pl.ds(j * LANES, LANES)] = (
                    srcv[pl.ds(base + j * LANES, LANES)] + tn)
            pltpu.async_copy(y_hbm.at[ibuf], rbuf, sem)

        gstart(0, idx0, rows0, gsem0)
        gstart(1, idx1, rows1, gsem1)

        def pair(pj, _):
            for p in range(2):
                ibuf, rbuf, gsem, sbo, ssem = bufs[p]
                b = 2 * pj + p
                base = b * KB
                pltpu.make_async_copy(y_hbm.at[ibuf], rbuf, gsem).wait()

                @pl.when(b >= 2)
                def _():
                    pltpu.make_async_copy(sbo, acc.at[dstv.at[b - 2]],
                                          ssem).wait()
                for j in range(KB // LANES):
                    nvec = ewv[pl.ds(base + j * LANES, LANES)]
                    for i in range(LANES):
                        nv = lax.broadcast_in_dim(nvec[i], (LANES,), ())
                        r = j * LANES + i
                        for q in range(W // LANES):
                            sbo[r, pl.ds(q * LANES, LANES)] = (
                                rbuf[r, pl.ds(q * LANES, LANES)] * nv)

                @pl.when(b + 2 < NB)
                def _():
                    gstart(b + 2, ibuf, rbuf, gsem)
                pltpu.async_copy(sbo, acc.at[dstv.at[b]], ssem, add=True)
            return 0
        lax.fori_loop(0, NB // 2, pair, 0)
        pltpu.make_async_copy(sout0, acc.at[dstv.at[NB - 2]], ssem0).wait()
        pltpu.make_async_copy(sout1, acc.at[dstv.at[NB - 1]], ssem1).wait()
        plsc.subcore_barrier()

        for k in range(ROWS_PER_TILE // SROWS):
            r0 = s * ROWS_PER_TILE + k * SROWS
            pltpu.sync_copy(acc.at[pl.ds(r0, SROWS), :], sbuf)
            pltpu.sync_copy(sbuf, p_hbm.at[t, pl.ds(r0, SROWS), :])

        @pl.when(s == NS - 1)
        def _():
            nrem = N - REM_OFF
            pltpu.sync_copy(acc.at[pl.ds(REM_OFF, nrem), :],
                            sbuf.at[pl.ds(0, nrem), :])
            pltpu.sync_copy(sbuf.at[pl.ds(0, nrem), :],
                            p_hbm.at[t, pl.ds(REM_OFF, nrem), :])
        plsc.subcore_barrier()
        return 0
    lax.fori_loop(0, TCH // NC, chunk, 0)


def _make_spmm_kernel(TCH, W=H):
    return pl.kernel(
        functools.partial(_spmm_body, W, TCH),
        out_type=jax.ShapeDtypeStruct((TCH, N, W), jnp.float32),
        mesh=_sc_mesh(),
        scratch_types=[
            pltpu.VMEM((CE,), jnp.int32),
            pltpu.VMEM((NB, KB), jnp.int32),
            pltpu.VMEM((CE,), jnp.float32),
            pltpu.VMEM((KB,), jnp.int32),
            pltpu.VMEM((KB,), jnp.int32),
            pltpu.VMEM((KB, W), jnp.float32),
            pltpu.VMEM((KB, W), jnp.float32),
            pltpu.VMEM((KB, W), jnp.float32),
            pltpu.VMEM((KB, W), jnp.float32),
            pltpu.VMEM((ZROWS, W), jnp.float32),
            pltpu.VMEM((SROWS, W), jnp.float32),
            pltpu.VMEM_SHARED((N, W), jnp.float32),
            pltpu.SemaphoreType.DMA,
            pltpu.SemaphoreType.DMA,
            pltpu.SemaphoreType.DMA,
            pltpu.SemaphoreType.DMA,
        ],
        compiler_params=pltpu.CompilerParams(use_tc_tiling_on_sc=False),
    )


_make_deg_kernel = functools.lru_cache(None)(_make_deg_kernel)
_make_spmm_kernel = functools.lru_cache(None)(_make_spmm_kernel)


def _deg_kernel(dst3, ew3):
    return _make_deg_kernel()(dst3, ew3)


def _spmm_f(*args):
    return _make_spmm_kernel(2 * T)(*args)


def _spmm_h(*args):
    return _make_spmm_kernel(T)(*args)


def _dinv_body(degp_ref, dinv_ref):
    deg = degp_ref[pl.ds(0, N)] + degp_ref[pl.ds(N, N)] + 1.0
    dinv_ref[...] = lax.rsqrt(jnp.maximum(deg, 1e-12))


def _dinv_kernel(degp):
    return pl.pallas_call(
        _dinv_body,
        out_shape=jax.ShapeDtypeStruct((N,), jnp.float32),
    )(degp)


def _prescale_body(x_ref, dinv_ref, y_ref):
    xx = x_ref[...] * dinv_ref[...][None]
    y_ref[:, 0] = xx[:, :, :H]
    y_ref[:, 1] = xx[:, :, H:]


def _prescale(X, dinv2, bn=1000):
    return pl.pallas_call(
        _prescale_body,
        grid=(N // bn,),
        in_specs=[pl.BlockSpec((T, bn, F), lambda i: (0, i, 0)),
                  pl.BlockSpec((bn, 1), lambda i: (i, 0))],
        out_specs=pl.BlockSpec((T, 2, bn, H), lambda i: (0, 0, i, 0)),
        out_shape=jax.ShapeDtypeStruct((T, 2, N, H), jnp.float32),
    )(X, dinv2)


def _layer_body(with_head, split, z_ref, y_ref, dinv_ref, wc_ref, bc_ref,
                wlt_ref, wlbzr_ref, wlbh_ref, blzr_ref, blh_ref,
                *rest):
    if with_head:
        (watt_ref, batt_ref, wout_ref, bout_ref, out_ref) = rest
    else:
        (out_ref,) = rest
    bn = dinv_ref.shape[0]
    dinv = dinv_ref[...]
    h = jnp.zeros((bn, H), jnp.float32)
    wc = wc_ref[...]
    bc = bc_ref[...]
    wlt = wlt_ref[...]
    wlbzr = wlbzr_ref[...]
    wlbh = wlbh_ref[...]
    blzr = blzr_ref[...]
    blh = blh_ref[...]
    hs = []
    es = []
    for t in range(T):
        if split:
            zt = jnp.concatenate([z_ref[2 * t], z_ref[2 * t + 1]], axis=1)
            yt = jnp.concatenate([y_ref[t, 0], y_ref[t, 1]], axis=1)
        else:
            zt = z_ref[t]
            yt = y_ref[t]
        conv = dinv * (zt + yt)
        c3 = jnp.dot(conv, wc, preferred_element_type=jnp.float32) + bc
        p3 = jnp.dot(c3, wlt, preferred_element_type=jnp.float32)
        hzr = jnp.dot(h, wlbzr, preferred_element_type=jnp.float32)
        zr = jax.nn.sigmoid(p3[:, :2 * H] + hzr + blzr)
        z = zr[:, :H]
        r = zr[:, H:]
        htil = jnp.tanh(p3[:, 2 * H:] +
                        jnp.dot(h * r, wlbh, preferred_element_type=jnp.float32)
                        + blh)
        h = z * h + (1.0 - z) * htil
        if with_head:
            hs.append(h)
            es.append(jnp.sum(h * watt_ref[...], axis=1) + batt_ref[0])
        else:
            out_ref[t] = h * dinv
    if with_head:
        e = jnp.stack(es, axis=0)
        m = jnp.max(e, axis=0, keepdims=True)
        pe = jnp.exp(e - m)
        att = pe / jnp.sum(pe, axis=0, keepdims=True)
        ctx = jnp.zeros((bn, H), jnp.float32)
        for t in range(T):
            ctx = ctx + att[t][:, None] * hs[t]
        out_ref[...] = (jnp.dot(ctx, wout_ref[...],
                                preferred_element_type=jnp.float32)
                        + bout_ref[...])


def _run_layer(Z, Y, dinv2, wc, bc, wlt, wlbzr, wlbh, blzr, blh,
               head=None, split=False, bn=400):
    grid = (N // bn,)
    full = lambda shape: pl.BlockSpec(shape, lambda i: (0,) * len(shape))
    if split:
        zspec = pl.BlockSpec((2 * T, bn, H), lambda i: (0, i, 0))
        yspec = pl.BlockSpec((T, 2, bn, H), lambda i: (0, 0, i, 0))
    else:
        zspec = pl.BlockSpec((T, bn, H), lambda i: (0, i, 0))
        yspec = pl.BlockSpec((T, bn, H), lambda i: (0, i, 0))
    in_specs = [
        zspec, yspec,
        pl.BlockSpec((bn, 1), lambda i: (i, 0)),
        full(wc.shape), full(bc.shape), full(wlt.shape),
        full(wlbzr.shape), full(wlbh.shape), full(blzr.shape),
        full(blh.shape),
    ]
    args = [Z, Y, dinv2, wc, bc, wlt, wlbzr, wlbh, blzr, blh]
    if head is None:
        out_shape = jax.ShapeDtypeStruct((T, N, H), jnp.float32)
        out_spec = pl.BlockSpec((T, bn, H), lambda i: (0, i, 0))
        body = functools.partial(_layer_body, False, split)
    else:
        watt, batt, wout, bout = head
        in_specs += [full(watt.shape), full(batt.shape), full(wout.shape),
                     full(bout.shape)]
        args += [watt, batt, wout, bout]
        out_shape = jax.ShapeDtypeStruct((N, OUT), jnp.float32)
        out_spec = pl.BlockSpec((bn, OUT), lambda i: (i, 0))
        body = functools.partial(_layer_body, True, split)
    return pl.pallas_call(
        body, grid=grid, in_specs=in_specs, out_specs=out_spec,
        out_shape=out_shape,
    )(*args)


def _gate_weights(lp):
    wc = jnp.concatenate([lp["Wc_z"], lp["Wc_r"], lp["Wc_h"]], axis=1)
    bc = jnp.concatenate([lp["bc_z"], lp["bc_r"], lp["bc_h"]])
    wlt = jax.scipy.linalg.block_diag(lp["Wl_z"][:H], lp["Wl_r"][:H],
                                      lp["Wl_h"][:H])
    wlbzr = jnp.concatenate([lp["Wl_z"][H:], lp["Wl_r"][H:]], axis=1)
    wlbh = lp["Wl_h"][H:]
    blzr = jnp.concatenate([lp["bl_z"], lp["bl_r"]])
    blh = lp["bl_h"]
    return wc, bc, wlt, wlbzr, wlbh, blzr, blh


def kernel(x, edge_index, edge_weight, params):
    src = edge_index[0].astype(jnp.int32)
    dst = edge_index[1].astype(jnp.int32)
    ew = edge_weight.astype(jnp.float32)

    degp = _deg_kernel(dst.reshape(NC * NS, NBD, KD),
                       ew.reshape(NC * NS, NBD, KD))
    dinv = _dinv_kernel(degp)
    dinv2 = dinv.reshape(N, 1)

    src2 = src.reshape(NS, CE)
    dst3 = dst.reshape(NS, NB, KB)
    ew2 = ew.reshape(NS, CE)

    xt = jnp.transpose(x, (2, 0, 1))
    Y0 = _prescale(xt, dinv2)
    Z1 = _spmm_f(Y0.reshape(2 * T * N, H), src2, dst3, ew2)

    w0 = _gate_weights(params["layers"][0])
    Y1 = _run_layer(Z1, Y0, dinv2, *w0, split=True)

    Z2 = _spmm_h(Y1.reshape(T * N, H), src2, dst3, ew2)

    w1 = _gate_weights(params["layers"][1])
    head = (params["W_att"].reshape(1, H), params["b_att"],
            params["W_out"], params["b_out"])
    return _run_layer(Z2, Y1, dinv2, *w1, head=head)
